# baseline (speedup 1.0000x reference)
import functools

import jax
import jax.numpy as jnp
from jax.experimental import pallas as pl

BLOCK_M = 4096


def _body(x_ref, w_ref, b_ref, o_ref):
    o_ref[...] = (
        jnp.dot(x_ref[...], w_ref[...], preferred_element_type=jnp.float32)
        + b_ref[...]
    )


@functools.partial(jax.jit, static_argnames=())
def kernel(features, indices, W, b):
    del indices
    m, c_in = features.shape
    c_out = W.shape[1]
    block_m = min(BLOCK_M, m)
    grid = (pl.cdiv(m, block_m),)
    return pl.pallas_call(
        _body,
        grid=grid,
        in_specs=[
            pl.BlockSpec((block_m, c_in), lambda i: (i, 0)),
            pl.BlockSpec((c_in, c_out), lambda i: (0, 0)),
            pl.BlockSpec((1, c_out), lambda i: (0, 0)),
        ],
        out_specs=pl.BlockSpec((block_m, c_out), lambda i: (i, 0)),
        out_shape=jax.ShapeDtypeStruct((m, c_out), jnp.float32),
    )(features, W, b.reshape(1, c_out))


# speedup vs baseline: 1.0917x; 1.0917x over previous
import functools

import jax
import jax.numpy as jnp
from jax.experimental import pallas as pl

BLOCK_M = 16384


def _body(x_ref, w_ref, b_ref, o_ref):
    o_ref[...] = (
        jnp.dot(x_ref[...], w_ref[...], preferred_element_type=jnp.float32)
        + b_ref[...]
    )


@functools.partial(jax.jit, static_argnames=())
def kernel(features, indices, W, b):
    del indices
    m, c_in = features.shape
    c_out = W.shape[1]
    block_m = min(BLOCK_M, m)
    grid = (pl.cdiv(m, block_m),)
    return pl.pallas_call(
        _body,
        grid=grid,
        in_specs=[
            pl.BlockSpec((block_m, c_in), lambda i: (i, 0)),
            pl.BlockSpec((c_in, c_out), lambda i: (0, 0)),
            pl.BlockSpec((1, c_out), lambda i: (0, 0)),
        ],
        out_specs=pl.BlockSpec((block_m, c_out), lambda i: (i, 0)),
        out_shape=jax.ShapeDtypeStruct((m, c_out), jnp.float32),
    )(features, W, b.reshape(1, c_out))
